# Initial kernel scaffold; baseline (speedup 1.0000x reference)
import jax
import jax.numpy as jnp
from jax.experimental import pallas as pl


def kernel(dr_vec, Z, neighbor_idxs, embeddings):
    raise NotImplementedError("write your pallas kernel here")


# speedup vs baseline: 13.0983x; 13.0983x over previous
import math

import jax
import jax.numpy as jnp
import numpy as np
from jax.experimental import pallas as pl
from jax.experimental.pallas import tpu as pltpu

_N_SPECIES = 119
_N_RADIAL = 5
_N_BASIS = 7
_R_MIN = 0.5
_R_MAX = 6.0

_P2 = [(i, j) for i in range(3) for j in range(i, 3)]
_P3 = [(i, j, k) for i in range(3) for j in range(i, 3) for k in range(j, 3)]
_P2_POS = {p: n for n, p in enumerate(_P2)}
_P3_POS = {p: n for n, p in enumerate(_P3)}
_W2 = np.array([1.0 if i == j else 2.0 for (i, j) in _P2], np.float32)
_W3 = np.array(
    [6.0 / (np.prod([math.factorial((i, j, k).count(v)) for v in set((i, j, k))]))
     for (i, j, k) in _P3], np.float32)

_T2 = [(r, s) for r in range(_N_RADIAL) for s in range(r, _N_RADIAL)]
_T3 = [(r, s, t) for r in range(_N_RADIAL) for s in range(r, _N_RADIAL)
       for t in range(s, _N_RADIAL)]

_C_M0 = 0
_C_M1 = 5
_C_M2 = 20
_C_M3 = 50
_NMOM = 100


def _m1_col(r, i):
    return _C_M1 + r * 3 + i


def _m2_col(r, i, j):
    return _C_M2 + r * 6 + _P2_POS[tuple(sorted((i, j)))]


def _m3_col(r, i, j, k):
    return _C_M3 + r * 10 + _P3_POS[tuple(sorted((i, j, k)))]


def _contract_body(mom_ref, out_ref):
    m = [mom_ref[c, 0] for c in range(_NMOM)]

    def m0(r):
        return m[_C_M0 + r]

    def m1(r, i):
        return m[_m1_col(r, i)]

    def m2(r, i, j):
        return m[_m2_col(r, i, j)]

    def m3(r, i, j, k):
        return m[_m3_col(r, i, j, k)]

    out = []
    for r in range(_N_RADIAL):
        out.append(m0(r))
    for (r, s) in _T2:
        out.append(sum(m1(r, i) * m1(s, i) for i in range(3)))
    for (r, s) in _T2:
        out.append(sum(_W2[p] * m2(r, i, j) * m2(s, i, j)
                       for p, (i, j) in enumerate(_P2)))
    for (r, s) in _T2:
        out.append(sum(_W3[p] * m3(r, i, j, k) * m3(s, i, j, k)
                       for p, (i, j, k) in enumerate(_P3)))
    b4 = {}
    for (r, s) in _T2:
        for j in range(3):
            for k in range(3):
                b4[(r, s, j, k)] = sum(m2(r, i, j) * m2(s, i, k) for i in range(3))
    for (r, s, t) in _T3:
        out.append(sum(b4[(r, s, j, k)] * m2(t, j, k)
                       for j in range(3) for k in range(3)))
    v5 = {}
    for s in range(_N_RADIAL):
        for t in range(_N_RADIAL):
            for i in range(3):
                v5[(s, t, i)] = sum(m1(s, j) * m2(t, i, j) for j in range(3))
    for r in range(_N_RADIAL):
        for s in range(_N_RADIAL):
            for t in range(_N_RADIAL):
                out.append(sum(m1(r, i) * v5[(s, t, i)] for i in range(3)))
    p6 = {}
    for (r, s) in _T2:
        for k in range(3):
            for l in range(3):
                p6[(r, s, k, l)] = sum(m3(r, i, j, k) * m3(s, i, j, l)
                                       for i in range(3) for j in range(3))
    for (r, s) in _T2:
        for t in range(_N_RADIAL):
            out.append(sum(p6[(r, s, k, l)] * m2(t, k, l)
                           for k in range(3) for l in range(3)))
    q7 = {}
    for r in range(_N_RADIAL):
        for s in range(_N_RADIAL):
            for k in range(3):
                q7[(r, s, k)] = sum(m3(r, i, j, k) * m2(s, i, j)
                                    for i in range(3) for j in range(3))
    for r in range(_N_RADIAL):
        for s in range(_N_RADIAL):
            for t in range(_N_RADIAL):
                out.append(sum(q7[(r, s, k)] * m1(t, k) for k in range(3)))

    for c, v in enumerate(out):
        out_ref[c, 0] = v


_N_OUT = 410


def _contract(mom_blocks):
    nb = mom_blocks.shape[1]
    return pl.pallas_call(
        _contract_body,
        grid=(nb,),
        in_specs=[pl.BlockSpec((_NMOM, 1, 8, 128), lambda b: (0, b, 0, 0))],
        out_specs=pl.BlockSpec((_N_OUT, 1, 8, 128), lambda b: (0, b, 0, 0)),
        out_shape=jax.ShapeDtypeStruct((_N_OUT, nb, 8, 128), jnp.float32),
    )(mom_blocks)


def _edge_moments(dr_vec, Z, neighbor_idxs, embeddings):
    n_atoms = Z.shape[0]
    idx_i = neighbor_idxs[0]
    idx_j = neighbor_idxs[1]
    dr = jnp.sqrt(jnp.sum(dr_vec ** 2, axis=-1))
    dn = dr_vec / (dr + 1e-05)[:, None]
    betta = _N_BASIS ** 2 / _R_MAX ** 2
    rad_norm = (2.0 * betta / np.pi) ** 0.25
    shifts = (_R_MIN + (_R_MAX - _R_MIN) / _N_BASIS * np.arange(_N_BASIS)).astype(np.float32)
    basis = rad_norm * jnp.exp(-betta * (shifts[None, :] - dr[:, None]) ** 2)
    embed_norm = 1.0 / np.sqrt(_N_BASIS)
    coeffs = embed_norm * embeddings[Z[idx_j], Z[idx_i]]
    radial = jnp.einsum("nrb,nb->nr", coeffs, basis)
    cutoff = 0.5 * (jnp.cos(np.pi * jnp.minimum(dr, _R_MAX) / _R_MAX) + 1.0)
    radial = radial * cutoff[:, None]

    mono2 = jnp.stack([dn[:, i] * dn[:, j] for (i, j) in _P2], axis=-1)
    mono3 = jnp.stack([dn[:, i] * dn[:, j] * dn[:, k] for (i, j, k) in _P3],
                      axis=-1)
    payload = jnp.concatenate([
        radial,
        (radial[:, :, None] * dn[:, None, :]).reshape(-1, 15),
        (radial[:, :, None] * mono2[:, None, :]).reshape(-1, 30),
        (radial[:, :, None] * mono3[:, None, :]).reshape(-1, 50),
    ], axis=-1)
    mom = jax.ops.segment_sum(payload, idx_j, num_segments=n_atoms)
    return mom


def kernel(dr_vec, Z, neighbor_idxs, embeddings):
    n_atoms = Z.shape[0]
    mom = _edge_moments(dr_vec, Z, neighbor_idxs, embeddings)
    nb = (n_atoms + 1023) // 1024
    pad = nb * 1024 - n_atoms
    mom_t = jnp.pad(mom, ((0, pad), (0, 0))).T.reshape(_NMOM, nb, 8, 128)
    out_t = _contract(mom_t)
    out = out_t.reshape(_N_OUT, nb * 1024).T[:n_atoms]
    return out


# speedup vs baseline: 34.3561x; 2.6229x over previous
import functools
import math

import jax
import jax.numpy as jnp
import numpy as np
from jax import lax
from jax.experimental import pallas as pl
from jax.experimental.pallas import tpu as pltpu
from jax.experimental.pallas import tpu_sc as plsc

_N_SPECIES = 119
_N_RADIAL = 5
_N_BASIS = 7
_R_MIN = 0.5
_R_MAX = 6.0

_E_PAD = 819200
_A_PAD = 50176
_CHUNK = 12544
_ACC_ROWS = 12672
_NPAIR = _N_SPECIES * _N_SPECIES
_NPAIR_PAD = 14168
_TABW = 128
_NMOM = 100
_MOMW = 128
_N_OUT = 410

_BETTA = _N_BASIS ** 2 / _R_MAX ** 2
_RAD_NORM = (2.0 * _BETTA / np.pi) ** 0.25
_EMBED_NORM = 1.0 / math.sqrt(_N_BASIS)
_SHIFTS = (_R_MIN + (_R_MAX - _R_MIN) / _N_BASIS * np.arange(_N_BASIS)).astype(np.float32)

_P2 = [(i, j) for i in range(3) for j in range(i, 3)]
_P3 = [(i, j, k) for i in range(3) for j in range(i, 3) for k in range(j, 3)]
_P2_POS = {p: n for n, p in enumerate(_P2)}
_P3_POS = {p: n for n, p in enumerate(_P3)}
_W2 = np.array([1.0 if i == j else 2.0 for (i, j) in _P2], np.float32)
_W3 = np.array(
    [6.0 / (np.prod([math.factorial((i, j, k).count(v)) for v in set((i, j, k))]))
     for (i, j, k) in _P3], np.float32)

_T2 = [(r, s) for r in range(_N_RADIAL) for s in range(r, _N_RADIAL)]
_T3 = [(r, s, t) for r in range(_N_RADIAL) for s in range(r, _N_RADIAL)
       for t in range(s, _N_RADIAL)]

_C_M1, _C_M2, _C_M3 = 5, 20, 50


def _m1_col(r, i):
    return _C_M1 + r * 3 + i


def _m2_col(r, i, j):
    return _C_M2 + r * 6 + _P2_POS[tuple(sorted((i, j)))]


def _m3_col(r, i, j, k):
    return _C_M3 + r * 10 + _P3_POS[tuple(sorted((i, j, k)))]


_A2_B = 1024
_A2_NB = _E_PAD // 32 // _A2_B
_A2_Q = 256

_COS_COEF = np.array([
    1.00000000e+00, -4.93480220e+00, 4.05871213e+00, -1.33526277e+00,
    2.35330627e-01, -2.58068799e-02, 1.92954737e-03, -1.04599095e-04,
    4.26890532e-06, -1.22090180e-07], np.float32)


def _rsqrt_nr(x):
    i = plsc.bitcast(x, jnp.int32)
    y = plsc.bitcast(jnp.int32(0x5F3759DF) - (i >> 1), jnp.float32)
    for _ in range(3):
        y = y * (1.5 - 0.5 * x * y * y)
    return y


def _a2_body(tab_hbm, z_hbm, idxi_hbm, idxj_hbm, drv_hbm,
             raddn_hbm, ii_v, ij_v, zi_v, zj_v, drv_v, pair_v,
             crow_v, out_v, sem):
    c = lax.axis_index("c")
    s = lax.axis_index("s")
    wid = c * 16 + s
    rows0 = jnp.arange(16, dtype=jnp.int32)
    lane3 = rows0 * 3

    def batch(b, carry):
        base = wid * (_A2_NB * _A2_B) + b * _A2_B
        pltpu.sync_copy(idxi_hbm.at[pl.ds(base, _A2_B)], ii_v)
        pltpu.sync_copy(idxj_hbm.at[pl.ds(base, _A2_B)], ij_v)
        pltpu.sync_copy(drv_hbm.at[pl.ds(base * 3, _A2_B * 3)], drv_v)
        pltpu.async_copy(z_hbm.at[ii_v], zi_v, sem).wait()
        pltpu.async_copy(z_hbm.at[ij_v], zj_v, sem).wait()

        def pairs(v, carryp):
            sl = pl.ds(v * 16, 16)
            pair_v[sl] = zj_v[sl] * _N_SPECIES + zi_v[sl]
            return carryp

        lax.fori_loop(0, _A2_B // 16, pairs, 0)

        for q in range(_A2_B // _A2_Q):
            pltpu.async_copy(
                tab_hbm.at[pair_v.at[pl.ds(q * _A2_Q, _A2_Q)]],
                crow_v, sem).wait()

            for h in range(_A2_Q // 128):
                r2 = (q * _A2_Q) // 128 + h

                def vblock(v, carryv, r2=r2, h=h, q=q):
                    e0 = q * _A2_Q + h * 128 + v * 16
                    rows = rows0 + (h * 128 + v * 16)
                    x = plsc.load_gather(drv_v, [lane3 + (e0 * 3)])
                    y = plsc.load_gather(drv_v, [lane3 + (e0 * 3 + 1)])
                    z = plsc.load_gather(drv_v, [lane3 + (e0 * 3 + 2)])
                    dr2 = x * x + y * y + z * z
                    dr = dr2 * _rsqrt_nr(dr2)
                    t = jnp.minimum(dr, _R_MAX) * np.float32(1.0 / _R_MAX)
                    u = t * t
                    cosv = jnp.full((16,), _COS_COEF[9], jnp.float32)
                    for k in range(8, -1, -1):
                        cosv = cosv * u + _COS_COEF[k]
                    scale = (0.5 * (cosv + 1.0)) * np.float32(
                        _RAD_NORM * _EMBED_NORM)
                    bb = []
                    for b2 in range(_N_BASIS):
                        d = _SHIFTS[b2] - dr
                        bb.append(jnp.exp(np.float32(-_BETTA) * d * d) * scale)
                    sl = pl.ds(v * 16, 16)
                    for r in range(_N_RADIAL):
                        acc = plsc.load_gather(
                            crow_v,
                            [rows, jnp.full((16,), r * 8, jnp.int32)]) * bb[0]
                        for b2 in range(1, _N_BASIS):
                            acc = acc + plsc.load_gather(
                                crow_v,
                                [rows, jnp.full((16,), r * 8 + b2, jnp.int32)]
                            ) * bb[b2]
                        out_v[r, r2, sl] = acc
                    invq = _rsqrt_nr(dr + 1e-05)
                    inv_dr = invq * invq
                    out_v[5, r2, sl] = x * inv_dr
                    out_v[6, r2, sl] = y * inv_dr
                    out_v[7, r2, sl] = z * inv_dr
                    return carryv

                lax.fori_loop(0, 8, vblock, 0)
        pltpu.sync_copy(
            out_v,
            raddn_hbm.at[:, pl.ds(pl.multiple_of(base // 128, 8),
                                  _A2_B // 128), :])
        return carry

    lax.fori_loop(0, _A2_NB, batch, 0)


def _a2(tab, Z, idx_i, idx_j, drv_flat):
    mesh = plsc.VectorSubcoreMesh(core_axis_name="c", subcore_axis_name="s")
    f = pl.kernel(
        _a2_body,
        out_type=jax.ShapeDtypeStruct((8, _E_PAD // 128, 128), jnp.float32),
        mesh=mesh,
        compiler_params=pltpu.CompilerParams(needs_layout_passes=False),
        scratch_types=[
            pltpu.VMEM((_A2_B,), jnp.int32),
            pltpu.VMEM((_A2_B,), jnp.int32),
            pltpu.VMEM((_A2_B,), jnp.int32),
            pltpu.VMEM((_A2_B,), jnp.int32),
            pltpu.VMEM((_A2_B * 3,), jnp.float32),
            pltpu.VMEM((_A2_B,), jnp.int32),
            pltpu.VMEM((_A2_Q, 128), jnp.float32),
            pltpu.VMEM((8, _A2_B // 128, 128), jnp.float32),
            pltpu.SemaphoreType.DMA,
        ],
    )
    return f(tab, Z, idx_i, idx_j, drv_flat)


_B_B = 1024
_B_S = 128
_B_NB = _E_PAD // 16 // _B_B
_TPT = _ACC_ROWS // 16
_DPT = _CHUNK // 16

_PAY = []
for _r in range(_N_RADIAL):
    _PAY.append((_r, 0))
for _r in range(_N_RADIAL):
    for _i in range(3):
        _PAY.append((_r, 1 + _i))
for _r in range(_N_RADIAL):
    for _p, (_i, _j) in enumerate(_P2):
        _PAY.append((_r, 4 + _p))
for _r in range(_N_RADIAL):
    for _p, (_i, _j, _k) in enumerate(_P3):
        _PAY.append((_r, 10 + _p))
assert len(_PAY) == _NMOM


def _b_body(idxj_hbm, raddn_hbm, mom_hbm,
            acc_sp, ij_v, rd_v, pay_v, sidx_v, sem):
    c = lax.axis_index("c")
    s = lax.axis_index("s")
    rows0 = jnp.arange(16, dtype=jnp.int32)
    z16 = jnp.zeros((16,), jnp.float32)

    def one_pass(p, carry):
        chunk = 2 * p + c
        lo = chunk * _CHUNK

        def zfill(i, carry0):
            for c8 in range(_MOMW // 16):
                pay_v[i, pl.ds(c8 * 16, 16)] = z16
            return carry0

        lax.fori_loop(0, _B_S, zfill, 0)

        def zero(i, carry0):
            pltpu.sync_copy(pay_v, acc_sp.at[pl.ds(s * _TPT + i * _B_S, _B_S)])
            return carry0

        lax.fori_loop(0, _TPT // _B_S, zero, 0)
        pltpu.sync_copy(pay_v.at[pl.ds(0, _TPT % _B_S)],
                        acc_sp.at[pl.ds(s * _TPT + (_TPT // _B_S) * _B_S,
                                        _TPT % _B_S)])
        plsc.subcore_barrier()

        def batch(b, carry2):
            base = s * (_B_NB * _B_B) + b * _B_B
            pltpu.sync_copy(idxj_hbm.at[pl.ds(base, _B_B)], ij_v)
            pltpu.sync_copy(
                raddn_hbm.at[:, pl.ds(pl.multiple_of(base // 128, 8),
                                      _B_B // 128), :], rd_v)
            for u in range(_B_B // _B_S):

                def vblock(v, carryv, u=u):
                    sl = pl.ds(u * _B_S + v * 16, 16)
                    sl2 = pl.ds(v * 16, 16)
                    eo = rows0 + (v * 16)
                    ij = ij_v[sl]
                    inb = (ij >= lo) & (ij < lo + _CHUNK)
                    sidx_v[sl2] = jnp.where(inb, ij - lo, _CHUNK + eo)
                    rad = [rd_v[r, u, sl2] for r in range(_N_RADIAL)]
                    x = rd_v[5, u, sl2]
                    y = rd_v[6, u, sl2]
                    z = rd_v[7, u, sl2]
                    xx, xy, xz = x * x, x * y, x * z
                    yy, yz, zz = y * y, y * z, z * z
                    mono = [None, x, y, z, xx, xy, xz, yy, yz, zz,
                            xx * x, xx * y, xx * z, xy * y, xy * z, xz * z,
                            yy * y, yy * z, yz * z, zz * z]
                    for col, (r, g) in enumerate(_PAY):
                        val = rad[r] if g == 0 else rad[r] * mono[g]
                        plsc.store_scatter(
                            pay_v, [eo, jnp.full((16,), col, jnp.int32)], val)
                    return carryv

                lax.fori_loop(0, _B_S // 16, vblock, 0)
                pltpu.sync_copy(pay_v, acc_sp.at[sidx_v], add=True)
            return carry2

        lax.fori_loop(0, _B_NB, batch, 0)
        plsc.subcore_barrier()
        pltpu.sync_copy(
            acc_sp.at[pl.ds(s * _DPT, _DPT)],
            mom_hbm.at[pl.ds(pl.multiple_of(chunk * _CHUNK + s * _DPT, 8),
                             _DPT)])
        plsc.subcore_barrier()
        return carry

    lax.fori_loop(0, 2, one_pass, 0)


def _b(idx_j, raddn):
    mesh = plsc.VectorSubcoreMesh(core_axis_name="c", subcore_axis_name="s")
    f = pl.kernel(
        _b_body,
        out_type=jax.ShapeDtypeStruct((_A_PAD, _MOMW), jnp.float32),
        mesh=mesh,
        compiler_params=pltpu.CompilerParams(needs_layout_passes=False),
        scratch_types=[
            pltpu.VMEM_SHARED((_ACC_ROWS, _MOMW), jnp.float32),
            pltpu.VMEM((_B_B,), jnp.int32),
            pltpu.VMEM((8, _B_B // 128, 128), jnp.float32),
            pltpu.VMEM((_B_S, _MOMW), jnp.float32),
            pltpu.VMEM((_B_S,), jnp.int32),
            pltpu.SemaphoreType.DMA,
        ],
    )
    return f(idx_j, raddn)


def _contract_body(mom_ref, out_ref):
    m = [mom_ref[col] for col in range(_NMOM)]

    def m0(r):
        return m[r]

    def m1(r, i):
        return m[_m1_col(r, i)]

    def m2(r, i, j):
        return m[_m2_col(r, i, j)]

    def m3(r, i, j, k):
        return m[_m3_col(r, i, j, k)]

    out = []
    for r in range(_N_RADIAL):
        out.append(m0(r))
    for (r, s) in _T2:
        out.append(sum(m1(r, i) * m1(s, i) for i in range(3)))
    for (r, s) in _T2:
        out.append(sum(_W2[p] * m2(r, i, j) * m2(s, i, j)
                       for p, (i, j) in enumerate(_P2)))
    for (r, s) in _T2:
        out.append(sum(_W3[p] * m3(r, i, j, k) * m3(s, i, j, k)
                       for p, (i, j, k) in enumerate(_P3)))
    b4 = {}
    for (r, s) in _T2:
        for j in range(3):
            for k in range(3):
                b4[(r, s, j, k)] = sum(m2(r, i, j) * m2(s, i, k) for i in range(3))
    for (r, s, t) in _T3:
        out.append(sum(b4[(r, s, j, k)] * m2(t, j, k)
                       for j in range(3) for k in range(3)))
    v5 = {}
    for s in range(_N_RADIAL):
        for t in range(_N_RADIAL):
            for i in range(3):
                v5[(s, t, i)] = sum(m1(s, j) * m2(t, i, j) for j in range(3))
    for r in range(_N_RADIAL):
        for s in range(_N_RADIAL):
            for t in range(_N_RADIAL):
                out.append(sum(m1(r, i) * v5[(s, t, i)] for i in range(3)))
    p6 = {}
    for (r, s) in _T2:
        for k in range(3):
            for l in range(3):
                p6[(r, s, k, l)] = sum(m3(r, i, j, k) * m3(s, i, j, l)
                                       for i in range(3) for j in range(3))
    for (r, s) in _T2:
        for t in range(_N_RADIAL):
            out.append(sum(p6[(r, s, k, l)] * m2(t, k, l)
                           for k in range(3) for l in range(3)))
    q7 = {}
    for r in range(_N_RADIAL):
        for s in range(_N_RADIAL):
            for k in range(3):
                q7[(r, s, k)] = sum(m3(r, i, j, k) * m2(s, i, j)
                                    for i in range(3) for j in range(3))
    for r in range(_N_RADIAL):
        for s in range(_N_RADIAL):
            for t in range(_N_RADIAL):
                out.append(sum(q7[(r, s, k)] * m1(t, k) for k in range(3)))

    for col, v in enumerate(out):
        out_ref[col] = v


def _contract(mom_blocks):
    nb = mom_blocks.shape[1] // 8
    return pl.pallas_call(
        _contract_body,
        grid=(nb,),
        in_specs=[pl.BlockSpec((_MOMW, 8, 128), lambda b: (0, b, 0))],
        out_specs=pl.BlockSpec((_N_OUT, 8, 128), lambda b: (0, b, 0)),
        out_shape=jax.ShapeDtypeStruct((_N_OUT, nb * 8, 128), jnp.float32),
    )(mom_blocks)


def kernel(dr_vec, Z, neighbor_idxs, embeddings):
    n_atoms = Z.shape[0]
    n_edges = dr_vec.shape[0]
    epad = _E_PAD - n_edges

    drv = jnp.pad(dr_vec, ((0, epad), (0, 0)), constant_values=1.0)
    idx_i = jnp.pad(neighbor_idxs[0], (0, epad))
    idx_j0 = jnp.pad(neighbor_idxs[1], (0, epad))
    idx_jb = jnp.pad(neighbor_idxs[1], (0, epad),
                     constant_values=np.int32(1 << 20))
    tab = jnp.pad(embeddings.reshape(_NPAIR, _N_RADIAL, _N_BASIS),
                  ((0, _NPAIR_PAD - _NPAIR), (0, 1), (0, 1))
                  ).reshape(_NPAIR_PAD, 48)
    tab = jnp.pad(tab, ((0, 0), (0, 128 - 48)))

    raddn = _a2(tab, Z, idx_i, idx_j0, drv.reshape(-1))
    mom = _b(idx_jb, raddn)

    mom_t = mom.T.reshape(_MOMW, _A_PAD // 128, 128)
    out_t = _contract(mom_t)
    return out_t.reshape(_N_OUT, _A_PAD).T[:n_atoms]
